# Initial kernel scaffold; baseline (speedup 1.0000x reference)
import jax
import jax.numpy as jnp
from jax.experimental import pallas as pl


def kernel(x):
    raise NotImplementedError("write your pallas kernel here")


# speedup vs baseline: 7.2823x; 7.2823x over previous
import functools

import jax
import jax.numpy as jnp
from jax import lax
from jax.experimental import pallas as pl
from jax.experimental.pallas import tpu as pltpu
from jax.experimental.pallas import tpu_sc as plsc

B = 1024
D = 100000
K = 5000

_NC = 2
_NS = 16
_NW = _NC * _NS
_ROWS_PER_W = B // _NW
_NBINS = 256
_L = 16


def _mono_key(v):
    u = lax.bitcast_convert_type(v, jnp.int32)
    neg = lax.shift_right_arithmetic(u, jnp.int32(31))
    flip = lax.bitwise_or(neg, jnp.int32(-(2 ** 31)))
    return lax.bitwise_xor(u, flip)


_mesh = plsc.VectorSubcoreMesh(core_axis_name="c", subcore_axis_name="s")


@functools.partial(
    pl.kernel,
    mesh=_mesh,
    compiler_params=pltpu.CompilerParams(needs_layout_passes=False),
    out_type=jax.ShapeDtypeStruct((B,), jnp.float32),
    scratch_types=[
        pltpu.VMEM((D,), jnp.float32),
        pltpu.VMEM((_NBINS * _L,), jnp.int32),
        pltpu.VMEM((_NBINS,), jnp.int32),
        pltpu.VMEM((_ROWS_PER_W,), jnp.float32),
    ],
)
def _sc_thresh(x_hbm, out_hbm, rowbuf, hist, totals, thrbuf):
    wid = lax.axis_index("s") * _NC + lax.axis_index("c")
    lanes = lax.iota(jnp.int32, _L)
    lane_off = lanes * _NBINS
    ones_i = jnp.ones((_L,), jnp.int32)
    zeros_i = jnp.zeros((_L,), jnp.int32)

    def row_body(i, _):
        row = wid * _ROWS_PER_W + i
        pltpu.sync_copy(x_hbm.at[row], rowbuf)

        def run_level(level, m, r):
            guard = m > _L
            shift = jnp.int32(24 - level * 8)
            nv = jnp.where(guard, (m + _L - 1) // _L, 0)

            def zbody(j, _):
                hist[pl.ds(j * _L, _L)] = zeros_i
                return 0
            lax.fori_loop(0, jnp.where(guard, _NBINS, 0), zbody, 0)

            def hbody(j, _):
                base = j * _L
                v = rowbuf[pl.ds(base, _L)]
                valid = (base + lanes) < m
                key = _mono_key(v)
                bin_i = lax.shift_right_logical(key, shift) & jnp.int32(0xFF)
                idx = bin_i + lane_off
                plsc.addupdate_scatter(hist, [idx], ones_i, mask=valid)
                return 0
            lax.fori_loop(0, nv, hbody, 0)

            def tbody(c, _):
                start = c * _L
                acc = zeros_i
                for l in range(_L):
                    acc = acc + hist[pl.ds(l * _NBINS + start, _L)]
                totals[pl.ds(start, _L)] = acc
                return 0
            lax.fori_loop(0, jnp.where(guard, _NBINS // _L, 0), tbody, 0)

            def sbody(j, carry):
                above, bsel, rsel = carry
                c = _NBINS // _L - 1 - j
                t = totals[pl.ds(c * _L, _L)]
                rv = lax.rev(t, (0,))
                cs = plsc.cumsum(rv)
                suf_above = lax.rev(above + cs - rv, (0,))
                hit = jnp.logical_and(suf_above < r, suf_above + t >= r)
                bin_vec = c * _L + lanes
                bsel2 = jnp.maximum(bsel, jnp.max(jnp.where(hit, bin_vec, -1)))
                rsel2 = jnp.maximum(rsel, jnp.max(
                    jnp.where(hit, r - suf_above, -1)))
                return above + jnp.sum(t), bsel2, rsel2
            _, bsel, r2 = lax.fori_loop(
                0, jnp.where(guard, _NBINS // _L, 0), sbody,
                (jnp.int32(0), jnp.int32(-1), jnp.int32(-1)))

            def cbody(j, off):
                base = j * _L
                v = rowbuf[pl.ds(base, _L)]
                valid = (base + lanes) < m
                key = _mono_key(v)
                bin_i = lax.shift_right_logical(key, shift) & jnp.int32(0xFF)
                sel = jnp.logical_and(valid, bin_i == bsel)
                cs = plsc.cumsum(jnp.where(sel, ones_i, zeros_i))
                pos = off + cs - 1
                plsc.store_scatter(rowbuf, [pos], v, mask=sel)
                return off + jnp.max(cs)
            m2 = lax.fori_loop(0, nv, cbody, jnp.int32(0))
            return jnp.where(guard, m2, m), jnp.where(guard, r2, r)

        m_f, r_f = jnp.int32(D), jnp.int32(K)
        for level in range(4):
            m_f, r_f = run_level(level, m_f, r_f)

        v = rowbuf[pl.ds(0, _L)]
        valid = lanes < m_f
        v = jnp.where(valid, v, jnp.float32(-jnp.inf))
        ks, _vs = plsc.sort_key_val(v, v, descending=True)
        pick = jnp.minimum(r_f - 1, _L - 1)
        t_val = jnp.sum(jnp.where(lanes == pick, ks, jnp.float32(0.0)))
        plsc.store_scatter(thrbuf, [jnp.full((_L,), i, jnp.int32)],
                           jnp.full((_L,), t_val, jnp.float32),
                           mask=lanes == 0)
        return 0

    lax.fori_loop(0, _ROWS_PER_W, row_body, 0)
    pltpu.sync_copy(thrbuf, out_hbm.at[pl.ds(wid * _ROWS_PER_W, _ROWS_PER_W)])


_MASK_ROWS = 8


def _mask_body(x_ref, t_ref, o_ref):
    xv = x_ref[...]
    t = t_ref[...]
    o_ref[...] = jnp.where(xv >= t, xv, jnp.float32(0.0))


@jax.jit
def kernel(x):
    th = _sc_thresh(x)
    th2 = th.reshape(B, 1)
    out = pl.pallas_call(
        _mask_body,
        grid=(B // _MASK_ROWS,),
        in_specs=[
            pl.BlockSpec((_MASK_ROWS, D), lambda i: (i, 0)),
            pl.BlockSpec((_MASK_ROWS, 1), lambda i: (i, 0)),
        ],
        out_specs=pl.BlockSpec((_MASK_ROWS, D), lambda i: (i, 0)),
        out_shape=jax.ShapeDtypeStruct((B, D), jnp.float32),
    )(x, th2)
    return out


# speedup vs baseline: 13.7698x; 1.8909x over previous
import functools

import jax
import jax.numpy as jnp
from jax import lax
from jax.experimental import pallas as pl
from jax.experimental.pallas import tpu as pltpu
from jax.experimental.pallas import tpu_sc as plsc

B = 1024
D = 100000
K = 5000

_NC = 2
_NS = 16
_NW = _NC * _NS
_ROWS_PER_W = B // _NW
_L = 16

_L0_BINS = 1024
_L0_SHIFT = 22
_U = 10
_NV0 = D // _L
_REFINE = ((14, 256), (6, 256), (0, 64))


def _mono_key(v):
    u = lax.bitcast_convert_type(v, jnp.int32)
    neg = lax.shift_right_arithmetic(u, jnp.int32(31))
    flip = lax.bitwise_or(neg, jnp.int32(-(2 ** 31)))
    return lax.bitwise_xor(u, flip)


_mesh = plsc.VectorSubcoreMesh(core_axis_name="c", subcore_axis_name="s")


@functools.partial(
    pl.kernel,
    mesh=_mesh,
    compiler_params=pltpu.CompilerParams(needs_layout_passes=False),
    out_type=jax.ShapeDtypeStruct((B,), jnp.float32),
    scratch_types=[
        pltpu.VMEM((D,), jnp.float32),
        pltpu.VMEM((_L0_BINS * _L,), jnp.int32),
        pltpu.VMEM((_L0_BINS,), jnp.int32),
        pltpu.VMEM((_ROWS_PER_W,), jnp.float32),
    ],
)
def _sc_thresh(x_hbm, out_hbm, rowbuf, hist, totals, thrbuf):
    wid = lax.axis_index("s") * _NC + lax.axis_index("c")
    lanes = lax.iota(jnp.int32, _L)
    ones_i = jnp.ones((_L,), jnp.int32)
    zeros_i = jnp.zeros((_L,), jnp.int32)
    lane_off0 = lanes * _L0_BINS

    def suffix_scan(nbins, r):
        def sbody(j, carry):
            above, bsel, rsel = carry
            c = nbins // _L - 1 - j
            t = totals[pl.ds(c * _L, _L)]
            rv = lax.rev(t, (0,))
            cs = plsc.cumsum(rv)
            suf_above = lax.rev(above + cs - rv, (0,))
            hit = jnp.logical_and(suf_above < r, suf_above + t >= r)
            bin_vec = c * _L + lanes
            bsel2 = jnp.maximum(bsel, jnp.max(jnp.where(hit, bin_vec, -1)))
            rsel2 = jnp.maximum(
                rsel, jnp.max(jnp.where(hit, r - suf_above, -1)))
            return above + jnp.sum(t), bsel2, rsel2
        _, bsel, rsel = lax.fori_loop(
            0, nbins // _L, sbody,
            (jnp.int32(0), jnp.int32(-1), jnp.int32(-1)))
        return bsel, rsel

    def sum_lanes(nbins, ntrip):
        def tbody(c, _):
            start = c * _L
            acc = zeros_i
            for l in range(_L):
                acc = acc + hist[pl.ds(l * nbins + start, _L)]
            totals[pl.ds(start, _L)] = acc
            return 0
        lax.fori_loop(0, ntrip, tbody, 0)

    def row_body(i, _):
        row = wid * _ROWS_PER_W + i
        pltpu.sync_copy(x_hbm.at[row], rowbuf)

        def z0body(j, _):
            base = j * (_L * 8)
            for u in range(8):
                hist[pl.ds(base + u * _L, _L)] = zeros_i
            return 0
        lax.fori_loop(0, _L0_BINS * _L // (_L * 8), z0body, 0)

        def h0body(j, _):
            base = j * (_L * _U)
            for u in range(_U):
                v = rowbuf[pl.ds(base + u * _L, _L)]
                key = _mono_key(v)
                bin_i = lax.shift_right_logical(key, jnp.int32(_L0_SHIFT))
                plsc.addupdate_scatter(hist, [bin_i + lane_off0], ones_i)
            return 0
        lax.fori_loop(0, _NV0 // _U, h0body, 0)

        sum_lanes(_L0_BINS, _L0_BINS // _L)
        bsel, r = suffix_scan(_L0_BINS, jnp.int32(K))

        def c0body(j, off):
            base = j * (_L * _U)
            vs, sels, css = [], [], []
            for u in range(_U):
                v = rowbuf[pl.ds(base + u * _L, _L)]
                key = _mono_key(v)
                bin_i = lax.shift_right_logical(key, jnp.int32(_L0_SHIFT))
                sel = bin_i == bsel
                cs = plsc.cumsum(jnp.where(sel, ones_i, zeros_i))
                vs.append(v)
                sels.append(sel)
                css.append(cs)
            for u in range(_U):
                pos = off + css[u] - 1
                plsc.store_scatter(rowbuf, [pos], vs[u], mask=sels[u])
                off = off + css[u][_L - 1]
            return off
        m = lax.fori_loop(0, _NV0 // _U, c0body, jnp.int32(0))

        for shift, nbins in _REFINE:
            guard = m > _L
            nv = jnp.where(guard, (m + _L - 1) // _L, 0)
            lane_off = lanes * nbins

            def zbody(j, _):
                hist[pl.ds(j * _L, _L)] = zeros_i
                return 0
            lax.fori_loop(0, jnp.where(guard, nbins, 0), zbody, 0)

            def hbody(j, _, shift=shift, nbins=nbins, lane_off=lane_off,
                      m=m):
                base = j * _L
                v = rowbuf[pl.ds(base, _L)]
                valid = (base + lanes) < m
                key = _mono_key(v)
                bin_i = (lax.shift_right_logical(key, jnp.int32(shift))
                         & jnp.int32(nbins - 1))
                plsc.addupdate_scatter(
                    hist, [bin_i + lane_off], ones_i, mask=valid)
                return 0
            lax.fori_loop(0, nv, hbody, 0)

            sum_lanes(nbins, jnp.where(guard, nbins // _L, 0))
            bsel2, r2 = suffix_scan(nbins, r)

            def cbody(j, off, shift=shift, nbins=nbins, bsel2=bsel2, m=m):
                base = j * _L
                v = rowbuf[pl.ds(base, _L)]
                valid = (base + lanes) < m
                key = _mono_key(v)
                bin_i = (lax.shift_right_logical(key, jnp.int32(shift))
                         & jnp.int32(nbins - 1))
                sel = jnp.logical_and(valid, bin_i == bsel2)
                cs = plsc.cumsum(jnp.where(sel, ones_i, zeros_i))
                plsc.store_scatter(rowbuf, [off + cs - 1], v, mask=sel)
                return off + cs[_L - 1]
            m2 = lax.fori_loop(0, nv, cbody, jnp.int32(0))

            m = jnp.where(guard, m2, m)
            r = jnp.where(guard, r2, r)

        v = rowbuf[pl.ds(0, _L)]
        valid = lanes < m
        v = jnp.where(valid, v, jnp.float32(-jnp.inf))
        ks, _vs = plsc.sort_key_val(v, v, descending=True)
        pick = jnp.minimum(r - 1, _L - 1)
        t_val = jnp.sum(jnp.where(lanes == pick, ks, jnp.float32(0.0)))
        plsc.store_scatter(thrbuf, [jnp.full((_L,), i, jnp.int32)],
                           jnp.full((_L,), t_val, jnp.float32),
                           mask=lanes == 0)
        return 0

    lax.fori_loop(0, _ROWS_PER_W, row_body, 0)
    pltpu.sync_copy(thrbuf, out_hbm.at[pl.ds(wid * _ROWS_PER_W, _ROWS_PER_W)])


_MASK_ROWS = 8


def _mask_body(x_ref, t_ref, o_ref):
    xv = x_ref[...]
    t = t_ref[...]
    o_ref[...] = jnp.where(xv >= t, xv, jnp.float32(0.0))


@jax.jit
def kernel(x):
    th = _sc_thresh(x)
    th2 = th.reshape(B, 1)
    out = pl.pallas_call(
        _mask_body,
        grid=(B // _MASK_ROWS,),
        in_specs=[
            pl.BlockSpec((_MASK_ROWS, D), lambda i: (i, 0)),
            pl.BlockSpec((_MASK_ROWS, 1), lambda i: (i, 0)),
        ],
        out_specs=pl.BlockSpec((_MASK_ROWS, D), lambda i: (i, 0)),
        out_shape=jax.ShapeDtypeStruct((B, D), jnp.float32),
    )(x, th2)
    return out


# speedup vs baseline: 31.2059x; 2.2663x over previous
import functools

import jax
import jax.numpy as jnp
from jax import lax
from jax.experimental import pallas as pl
from jax.experimental.pallas import tpu as pltpu
from jax.experimental.pallas import tpu_sc as plsc

B = 1024
D = 100000
K = 5000

_NC = 2
_NS = 16
_NW = _NC * _NS
_ROWS_PER_W = B // _NW
_L = 16

_L0_BINS = 1024
_L0_SHIFT = 22
_NV0 = D // _L
_REFINE = ((14, 256), (6, 256), (0, 64))


def _mono_key(v):
    u = lax.bitcast_convert_type(v, jnp.int32)
    neg = lax.shift_right_arithmetic(u, jnp.int32(31))
    flip = lax.bitwise_or(neg, jnp.int32(-(2 ** 31)))
    return lax.bitwise_xor(u, flip)


def _bin10(v, extra):
    u = lax.bitcast_convert_type(v, jnp.int32)
    neg = lax.shift_right_arithmetic(u, jnp.int32(31))
    f = lax.bitwise_or(lax.bitwise_and(neg, jnp.int32(0x1FF)), extra)
    t = lax.shift_right_logical(u, jnp.int32(_L0_SHIFT))
    return lax.bitwise_xor(t, f)


_mesh = plsc.VectorSubcoreMesh(core_axis_name="c", subcore_axis_name="s")


@functools.partial(
    pl.kernel,
    mesh=_mesh,
    compiler_params=pltpu.CompilerParams(needs_layout_passes=False),
    out_type=jax.ShapeDtypeStruct((B,), jnp.float32),
    scratch_types=[
        pltpu.VMEM((D,), jnp.float32),
        pltpu.VMEM((_L0_BINS * _L,), jnp.int32),
        pltpu.VMEM((_L0_BINS,), jnp.int32),
        pltpu.VMEM((_ROWS_PER_W,), jnp.float32),
    ],
)
def _sc_thresh(x_hbm, out_hbm, rowbuf, hist, totals, thrbuf):
    wid = lax.axis_index("s") * _NC + lax.axis_index("c")
    lanes = lax.iota(jnp.int32, _L)
    ones_i = jnp.ones((_L,), jnp.int32)
    zeros_i = jnp.zeros((_L,), jnp.int32)
    l0_base = lax.bitwise_or(lanes * _L0_BINS, jnp.int32(0x200))
    l0_cmp = jnp.full((_L,), 0x200, jnp.int32)

    def suffix_scan(nbins, r):
        def sbody(j, carry):
            above, bsel, rsel = carry
            c = nbins // _L - 1 - j
            t = totals[pl.ds(c * _L, _L)]
            rv = lax.rev(t, (0,))
            cs = plsc.cumsum(rv)
            suf_above = lax.rev(above + cs - rv, (0,))
            hit = jnp.logical_and(suf_above < r, suf_above + t >= r)
            bin_vec = c * _L + lanes
            bsel2 = jnp.maximum(bsel, jnp.max(jnp.where(hit, bin_vec, -1)))
            rsel2 = jnp.maximum(
                rsel, jnp.max(jnp.where(hit, r - suf_above, -1)))
            return above + jnp.sum(t), bsel2, rsel2
        _, bsel, rsel = lax.fori_loop(
            0, nbins // _L, sbody,
            (jnp.int32(0), jnp.int32(-1), jnp.int32(-1)))
        return bsel, rsel

    def sum_lanes(nbins, ntrip):
        @plsc.parallel_loop(0, ntrip)
        def tbody(c):
            start = c * _L
            accs = [hist[pl.ds(l * nbins + start, _L)] for l in range(_L)]
            while len(accs) > 1:
                accs = [a + b for a, b in zip(accs[::2], accs[1::2])]
            totals[pl.ds(start, _L)] = accs[0]

    def row_body(i, _):
        row = wid * _ROWS_PER_W + i
        pltpu.sync_copy(x_hbm.at[row], rowbuf)

        @plsc.parallel_loop(0, _L0_BINS, unroll=8)
        def z0body(j):
            hist[pl.ds(j * _L, _L)] = zeros_i

        @plsc.parallel_loop(0, _NV0, unroll=8)
        def h0body(j):
            v = rowbuf[pl.ds(j * _L, _L)]
            plsc.addupdate_scatter(hist, [_bin10(v, l0_base)], ones_i)

        sum_lanes(_L0_BINS, _L0_BINS // _L)
        bsel, r = suffix_scan(_L0_BINS, jnp.int32(K))

        @plsc.parallel_loop(0, _NV0, unroll=4, carry=jnp.int32(0))
        def m(j, off):
            v = rowbuf[pl.ds(j * _L, _L)]
            sel = _bin10(v, l0_cmp) == bsel
            plsc.store_compressed(rowbuf.at[pl.ds(off, _L)], v, mask=sel)
            cnt = plsc.all_reduce_population_count(sel)
            return off + cnt[0]

        for shift, nbins in _REFINE:
            guard = m > _L
            nv = jnp.where(guard, (m + _L - 1) // _L, 0)
            lane_off = lanes * nbins

            @plsc.parallel_loop(0, jnp.where(guard, nbins, 0), unroll=8)
            def zbody(j):
                hist[pl.ds(j * _L, _L)] = zeros_i

            @plsc.parallel_loop(0, nv, unroll=2)
            def hbody(j, shift=shift, nbins=nbins, lane_off=lane_off, m=m):
                base = j * _L
                v = rowbuf[pl.ds(base, _L)]
                valid = (base + lanes) < m
                key = _mono_key(v)
                d = (lax.shift_right_logical(key, jnp.int32(shift))
                     & jnp.int32(nbins - 1))
                plsc.addupdate_scatter(
                    hist, [d + lane_off], ones_i, mask=valid)

            sum_lanes(nbins, jnp.where(guard, nbins // _L, 0))
            bsel2, r2 = suffix_scan(nbins, r)

            @plsc.parallel_loop(0, nv, unroll=2, carry=jnp.int32(0))
            def m2(j, off, shift=shift, nbins=nbins, bsel2=bsel2, m=m):
                base = j * _L
                v = rowbuf[pl.ds(base, _L)]
                valid = (base + lanes) < m
                key = _mono_key(v)
                d = (lax.shift_right_logical(key, jnp.int32(shift))
                     & jnp.int32(nbins - 1))
                sel = jnp.logical_and(valid, d == bsel2)
                plsc.store_compressed(rowbuf.at[pl.ds(off, _L)], v, mask=sel)
                cnt = plsc.all_reduce_population_count(sel)
                return off + cnt[0]

            m = jnp.where(guard, m2, m)
            r = jnp.where(guard, r2, r)

        v = rowbuf[pl.ds(0, _L)]
        valid = lanes < m
        v = jnp.where(valid, v, jnp.float32(-jnp.inf))
        ks, _vs = plsc.sort_key_val(v, v, descending=True)
        pick = jnp.minimum(r - 1, _L - 1)
        t_val = jnp.sum(jnp.where(lanes == pick, ks, jnp.float32(0.0)))
        plsc.store_scatter(thrbuf, [jnp.full((_L,), i, jnp.int32)],
                           jnp.full((_L,), t_val, jnp.float32),
                           mask=lanes == 0)
        return 0

    lax.fori_loop(0, _ROWS_PER_W, row_body, 0)
    pltpu.sync_copy(thrbuf, out_hbm.at[pl.ds(wid * _ROWS_PER_W, _ROWS_PER_W)])


_MASK_ROWS = 8


def _mask_body(x_ref, t_ref, o_ref):
    xv = x_ref[...]
    t = t_ref[...]
    o_ref[...] = jnp.where(xv >= t, xv, jnp.float32(0.0))


@jax.jit
def kernel(x):
    th = _sc_thresh(x)
    th2 = th.reshape(B, 1)
    out = pl.pallas_call(
        _mask_body,
        grid=(B // _MASK_ROWS,),
        in_specs=[
            pl.BlockSpec((_MASK_ROWS, D), lambda i: (i, 0)),
            pl.BlockSpec((_MASK_ROWS, 1), lambda i: (i, 0)),
        ],
        out_specs=pl.BlockSpec((_MASK_ROWS, D), lambda i: (i, 0)),
        out_shape=jax.ShapeDtypeStruct((B, D), jnp.float32),
    )(x, th2)
    return out
